# baseline (device time: 29636 ns/iter reference)
import jax
import jax.numpy as jnp
from jax import lax
from jax.experimental import pallas as pl
from jax.experimental.pallas import tpu as pltpu

N_DEV = 4
B, SQ, SKV, HQ, DH = 2, 256, 256, 16, 64
H_PER = HQ // N_DEV
DM = 512


def kernel(x, Wq, K_ext, V_ext, Wo):
    def body(x_ref, wq_ref, k_hbm, v_hbm, wo_ref, out_ref,
             k_vmem, v_vmem, comm_ref, send_sems, recv_sems, dma_sems):
        my_pos = lax.axis_index("i")
        p0 = jnp.bitwise_xor(my_pos, 1)
        p1 = jnp.bitwise_xor(my_pos, 2)
        h0 = my_pos * H_PER

        barrier_sem = pltpu.get_barrier_semaphore()
        for p in (p0, p1):
            pl.semaphore_signal(barrier_sem, inc=1, device_id=(p,),
                                device_id_type=pl.DeviceIdType.MESH)

        k_dma = pltpu.make_async_copy(
            k_hbm.at[:, :, pl.ds(h0, H_PER), :], k_vmem, dma_sems.at[0])
        v_dma = pltpu.make_async_copy(
            v_hbm.at[:, :, pl.ds(h0, H_PER), :], v_vmem, dma_sems.at[1])
        k_dma.start()
        v_dma.start()

        xf = x_ref[...].astype(jnp.bfloat16).reshape(B * SQ, DM)
        wqb = wq_ref[...].astype(jnp.bfloat16)
        q = jnp.dot(xf, wqb, preferred_element_type=jnp.float32)
        q = q.astype(jnp.bfloat16).reshape(B, SQ, H_PER, DH)
        wob = wo_ref[...].astype(jnp.bfloat16)

        qi = lax.broadcasted_iota(jnp.int32, (SQ, SKV), 0)
        ki = lax.broadcasted_iota(jnp.int32, (SQ, SKV), 1)
        mask = (jnp.abs(qi - ki) <= 128) | (ki < 32) | (qi < 32)

        k_dma.wait()
        v_dma.wait()

        def make_rdma(stage, b, partner):
            return pltpu.make_async_remote_copy(
                src_ref=comm_ref.at[2 * stage, b],
                dst_ref=comm_ref.at[2 * stage + 1, b],
                send_sem=send_sems.at[stage, b],
                recv_sem=recv_sems.at[stage, b],
                device_id=(partner,),
                device_id_type=pl.DeviceIdType.MESH,
            )

        rdma0 = [make_rdma(0, b, p0) for b in range(B)]
        rdma1 = [make_rdma(1, b, p1) for b in range(B)]

        barrier_waited = False
        for b in range(B):
            kb = k_vmem[b].astype(jnp.bfloat16)
            vb = v_vmem[b].astype(jnp.bfloat16)
            ctx_heads = []
            for h in range(H_PER):
                qb = q[b, :, h, :]
                s = lax.dot_general(
                    qb, kb[:, h, :], (((1,), (1,)), ((), ())),
                    preferred_element_type=jnp.float32,
                ) * 0.125
                s = jnp.where(mask, s, -1e9)
                s = s - s.max(axis=-1, keepdims=True)
                w = jnp.exp(s)
                w = w / w.sum(axis=-1, keepdims=True)
                ctx_heads.append(
                    jnp.dot(w.astype(jnp.bfloat16), vb[:, h, :],
                            preferred_element_type=jnp.float32))
            ctx = jnp.concatenate(ctx_heads, axis=1).astype(jnp.bfloat16)
            part = jnp.dot(ctx, wob, preferred_element_type=jnp.float32)
            out_ref[b] = part
            comm_ref[0, b] = part.astype(jnp.bfloat16)
            if not barrier_waited:
                pl.semaphore_wait(barrier_sem, 2)
                barrier_waited = True
            rdma0[b].start()

        for b in range(B):
            rdma0[b].wait_recv()
            acc = out_ref[b] + comm_ref[1, b].astype(jnp.float32)
            out_ref[b] = acc
            comm_ref[2, b] = acc.astype(jnp.bfloat16)
            rdma1[b].start()

        for b in range(B):
            rdma1[b].wait_recv()
            out_ref[b] = out_ref[b] + comm_ref[3, b].astype(jnp.float32)

        for b in range(B):
            rdma0[b].wait_send()
            rdma1[b].wait_send()

    out_shape = jax.ShapeDtypeStruct((B, SQ, DM), jnp.float32)
    return pl.pallas_call(
        body,
        out_shape=out_shape,
        in_specs=[
            pl.BlockSpec(memory_space=pltpu.VMEM),
            pl.BlockSpec(memory_space=pltpu.VMEM),
            pl.BlockSpec(memory_space=pltpu.MemorySpace.HBM),
            pl.BlockSpec(memory_space=pltpu.MemorySpace.HBM),
            pl.BlockSpec(memory_space=pltpu.VMEM),
        ],
        out_specs=pl.BlockSpec(memory_space=pltpu.VMEM),
        scratch_shapes=[
            pltpu.VMEM((B, SQ, H_PER, DH), jnp.float32),
            pltpu.VMEM((B, SQ, H_PER, DH), jnp.float32),
            pltpu.VMEM((4, B, SQ, DM), jnp.bfloat16),
            pltpu.SemaphoreType.DMA((2, B)),
            pltpu.SemaphoreType.DMA((2, B)),
            pltpu.SemaphoreType.DMA((2,)),
        ],
        compiler_params=pltpu.CompilerParams(collective_id=0),
    )(x, Wq, K_ext, V_ext, Wo)


# device time: 23452 ns/iter; 1.2637x vs baseline; 1.2637x over previous
import jax
import jax.numpy as jnp
from jax import lax
from jax.experimental import pallas as pl
from jax.experimental.pallas import tpu as pltpu

N_DEV = 4
B, SQ, SKV, HQ, DH = 2, 256, 256, 16, 64
H_PER = HQ // N_DEV
DM = 512
NC = 2
CS = SQ // NC


def kernel(x, Wq, K_ext, V_ext, Wo):
    my_i = lax.axis_index("i")
    h0 = my_i * H_PER
    K_sh = lax.dynamic_slice_in_dim(K_ext, h0, H_PER, axis=2).astype(jnp.bfloat16)
    V_sh = lax.dynamic_slice_in_dim(V_ext, h0, H_PER, axis=2).astype(jnp.bfloat16)
    xb = x.astype(jnp.bfloat16)
    Wqb = Wq.astype(jnp.bfloat16)
    Wob = Wo.astype(jnp.bfloat16)

    def body(x_ref, wq_ref, k_ref, v_ref, wo_ref, out_ref,
             comm_ref, send_sems, recv_sems):
        my_pos = lax.axis_index("i")
        pd = jnp.bitwise_xor(my_pos, 2)
        pn = jnp.bitwise_xor(my_pos, 1)

        barrier_sem = pltpu.get_barrier_semaphore()
        for p in (pd, pn):
            pl.semaphore_signal(barrier_sem, inc=1, device_id=(p,),
                                device_id_type=pl.DeviceIdType.MESH)

        xf = x_ref[...].reshape(B * SQ, DM)
        q = jnp.dot(xf, wq_ref[...], preferred_element_type=jnp.float32)
        q = q.astype(jnp.bfloat16).reshape(B, SQ, H_PER, DH)
        wob = wo_ref[...]

        qi = lax.broadcasted_iota(jnp.int32, (SQ, SKV), 0)
        ki = lax.broadcasted_iota(jnp.int32, (SQ, SKV), 1)
        mask = (jnp.abs(qi - ki) <= 128) | (ki < 32) | (qi < 32)

        def mk(stage, b, c, partner):
            sl = pl.ds(c * CS, CS)
            return pltpu.make_async_remote_copy(
                src_ref=comm_ref.at[2 * stage, b, sl],
                dst_ref=comm_ref.at[2 * stage + 1, b, sl],
                send_sem=send_sems.at[stage, b, c],
                recv_sem=recv_sems.at[stage, b, c],
                device_id=(partner,),
                device_id_type=pl.DeviceIdType.MESH,
            )

        rdma0 = [[mk(0, b, c, pd) for c in range(NC)] for b in range(B)]
        rdma1 = [[mk(1, b, c, pn) for c in range(NC)] for b in range(B)]

        barrier_waited = False
        for b in range(B):
            kb = k_ref[b]
            vb = v_ref[b]
            ctx_heads = []
            for h in range(H_PER):
                s = lax.dot_general(
                    q[b, :, h, :], kb[:, h, :], (((1,), (1,)), ((), ())),
                    preferred_element_type=jnp.float32,
                ) * 0.125
                w = jnp.where(mask, jnp.exp(s), 0.0)
                recip = 1.0 / w.sum(axis=-1, keepdims=True)
                ctx_heads.append(
                    jnp.dot(w.astype(jnp.bfloat16), vb[:, h, :],
                            preferred_element_type=jnp.float32) * recip)
            ctx = jnp.concatenate(ctx_heads, axis=1).astype(jnp.bfloat16)
            part = jnp.dot(ctx, wob, preferred_element_type=jnp.float32)
            out_ref[b] = part
            comm_ref[0, b] = part.astype(jnp.bfloat16)
            if not barrier_waited:
                pl.semaphore_wait(barrier_sem, 2)
                barrier_waited = True
            for c in range(NC):
                rdma0[b][c].start()

        for b in range(B):
            for c in range(NC):
                sl = pl.ds(c * CS, CS)
                rdma0[b][c].wait_recv()
                acc = out_ref[b, sl] + comm_ref[1, b, sl].astype(jnp.float32)
                out_ref[b, sl] = acc
                comm_ref[2, b, sl] = acc.astype(jnp.bfloat16)
                rdma1[b][c].start()

        for b in range(B):
            for c in range(NC):
                sl = pl.ds(c * CS, CS)
                rdma1[b][c].wait_recv()
                out_ref[b, sl] = (out_ref[b, sl]
                                  + comm_ref[3, b, sl].astype(jnp.float32))

        for b in range(B):
            for c in range(NC):
                rdma0[b][c].wait_send()
                rdma1[b][c].wait_send()

    out_shape = jax.ShapeDtypeStruct((B, SQ, DM), jnp.float32)
    return pl.pallas_call(
        body,
        out_shape=out_shape,
        in_specs=[pl.BlockSpec(memory_space=pltpu.VMEM)] * 5,
        out_specs=pl.BlockSpec(memory_space=pltpu.VMEM),
        scratch_shapes=[
            pltpu.VMEM((4, B, SQ, DM), jnp.bfloat16),
            pltpu.SemaphoreType.DMA((2, B, NC)),
            pltpu.SemaphoreType.DMA((2, B, NC)),
        ],
        compiler_params=pltpu.CompilerParams(collective_id=0),
    )(xb, Wqb, K_sh, V_sh, Wob)


# device time: 22040 ns/iter; 1.3446x vs baseline; 1.0641x over previous
import os

import jax
import jax.numpy as jnp
from jax import lax
from jax.experimental import pallas as pl
from jax.experimental.pallas import tpu as pltpu

_NO_COMM = os.environ.get("KNC") == "1"
_NO_COMPUTE = os.environ.get("KNP") == "1"

N_DEV = 4
B, SQ, SKV, HQ, DH = 2, 256, 256, 16, 64
H_PER = HQ // N_DEV
DM = 512
NC = 2
CS = SQ // NC


def kernel(x, Wq, K_ext, V_ext, Wo):
    my_i = lax.axis_index("i")
    h0 = my_i * H_PER
    K_sh = lax.dynamic_slice_in_dim(K_ext, h0, H_PER, axis=2)
    V_sh = lax.dynamic_slice_in_dim(V_ext, h0, H_PER, axis=2)

    def body(x_ref, wq_ref, k_ref, v_ref, wo_ref, out_ref,
             comm_ref, send_sems, recv_sems):
        my_pos = lax.axis_index("i")
        pd = jnp.bitwise_xor(my_pos, 2)
        pn = jnp.bitwise_xor(my_pos, 1)

        barrier_sem = pltpu.get_barrier_semaphore()
        for p in (pd, pn):
            pl.semaphore_signal(barrier_sem, inc=1, device_id=(p,),
                                device_id_type=pl.DeviceIdType.MESH)

        xf = x_ref[...].astype(jnp.bfloat16).reshape(B * SQ, DM)
        q = jnp.dot(xf, wq_ref[...].astype(jnp.bfloat16),
                    preferred_element_type=jnp.float32)
        q = q.astype(jnp.bfloat16).reshape(B, SQ, H_PER, DH)
        wob = wo_ref[...].astype(jnp.bfloat16)

        qi = lax.broadcasted_iota(jnp.int32, (SQ, SKV), 0)
        ki = lax.broadcasted_iota(jnp.int32, (SQ, SKV), 1)
        mask = (jnp.abs(qi - ki) <= 128) | (ki < 32) | (qi < 32)

        def mk(stage, b, c, partner):
            sl = pl.ds(c * CS, CS)
            return pltpu.make_async_remote_copy(
                src_ref=comm_ref.at[2 * stage, b, sl],
                dst_ref=comm_ref.at[2 * stage + 1, b, sl],
                send_sem=send_sems.at[stage, b, c],
                recv_sem=recv_sems.at[stage, b, c],
                device_id=(partner,),
                device_id_type=pl.DeviceIdType.MESH,
            )

        rdma0 = [[mk(0, b, c, pd) for c in range(NC)] for b in range(B)]
        rdma1 = [[mk(1, b, c, pn) for c in range(NC)] for b in range(B)]

        barrier_waited = False
        for b in range(B):
            if _NO_COMPUTE:
                part = x_ref[b].astype(jnp.float32)
            else:
                kb = k_ref[b].astype(jnp.bfloat16)
                vb = v_ref[b].astype(jnp.bfloat16)
                ctx_heads = []
                for h in range(H_PER):
                    s = lax.dot_general(
                        q[b, :, h, :], kb[:, h, :], (((1,), (1,)), ((), ())),
                        preferred_element_type=jnp.float32,
                    ) * 0.125
                    w = jnp.where(mask, jnp.exp(s), 0.0)
                    recip = 1.0 / w.sum(axis=-1, keepdims=True)
                    ctx_heads.append(
                        jnp.dot(w.astype(jnp.bfloat16), vb[:, h, :],
                                preferred_element_type=jnp.float32) * recip)
                ctx = jnp.concatenate(ctx_heads, axis=1).astype(jnp.bfloat16)
                part = jnp.dot(ctx, wob, preferred_element_type=jnp.float32)
            out_ref[b] = part
            comm_ref[0, b] = part.astype(jnp.bfloat16)
            if not barrier_waited:
                pl.semaphore_wait(barrier_sem, 2)
                barrier_waited = True
            if not _NO_COMM:
                for c in range(NC):
                    rdma0[b][c].start()

        if _NO_COMM:
            return
        for b in range(B):
            for c in range(NC):
                sl = pl.ds(c * CS, CS)
                rdma0[b][c].wait_recv()
                acc = out_ref[b, sl] + comm_ref[1, b, sl].astype(jnp.float32)
                out_ref[b, sl] = acc
                comm_ref[2, b, sl] = acc.astype(jnp.bfloat16)
                rdma1[b][c].start()

        for b in range(B):
            for c in range(NC):
                sl = pl.ds(c * CS, CS)
                rdma1[b][c].wait_recv()
                out_ref[b, sl] = (out_ref[b, sl]
                                  + comm_ref[3, b, sl].astype(jnp.float32))

        for b in range(B):
            for c in range(NC):
                rdma0[b][c].wait_send()
                rdma1[b][c].wait_send()

    out_shape = jax.ShapeDtypeStruct((B, SQ, DM), jnp.float32)
    return pl.pallas_call(
        body,
        out_shape=out_shape,
        in_specs=[pl.BlockSpec(memory_space=pltpu.VMEM)] * 5,
        out_specs=pl.BlockSpec(memory_space=pltpu.VMEM),
        scratch_shapes=[
            pltpu.VMEM((4, B, SQ, DM), jnp.bfloat16),
            pltpu.SemaphoreType.DMA((2, B, NC)),
            pltpu.SemaphoreType.DMA((2, B, NC)),
        ],
        compiler_params=pltpu.CompilerParams(collective_id=0),
    )(x, Wq, K_sh, V_sh, Wo)


# device time: 21358 ns/iter; 1.3876x vs baseline; 1.0319x over previous
import os

import jax
import jax.numpy as jnp
from jax import lax
from jax.experimental import pallas as pl
from jax.experimental.pallas import tpu as pltpu

_NO_COMM = os.environ.get("KNC") == "1"
_NO_COMPUTE = os.environ.get("KNP") == "1"

N_DEV = 4
B, SQ, SKV, HQ, DH = 2, 256, 256, 16, 64
H_PER = HQ // N_DEV
DM = 512
NBF = 2
CH = SQ // NBF


def kernel(x, Wq, K_ext, V_ext, Wo):
    my_i = lax.axis_index("i")
    h0 = my_i * H_PER
    K_sh = lax.dynamic_slice_in_dim(K_ext, h0, H_PER, axis=2)
    V_sh = lax.dynamic_slice_in_dim(V_ext, h0, H_PER, axis=2)

    def body(x_ref, wq_ref, k_ref, v_ref, wo_ref, out_ref,
             comm_ref, send_sems, recv_sems):
        my_pos = lax.axis_index("i")
        pn = jnp.bitwise_xor(my_pos, 1)
        pd = jnp.bitwise_xor(my_pos, 2)

        barrier_sem = pltpu.get_barrier_semaphore()
        for p in (pn, pd):
            pl.semaphore_signal(barrier_sem, inc=1, device_id=(p,),
                                device_id_type=pl.DeviceIdType.MESH)

        xf = x_ref[...].astype(jnp.bfloat16).reshape(B * SQ, DM)
        q = jnp.dot(xf, wq_ref[...].astype(jnp.bfloat16),
                    preferred_element_type=jnp.float32)
        q = q.astype(jnp.bfloat16).reshape(B, SQ, H_PER, DH)
        wob = wo_ref[...].astype(jnp.bfloat16)

        qi = lax.broadcasted_iota(jnp.int32, (SQ, SKV), 0)
        ki = lax.broadcasted_iota(jnp.int32, (SQ, SKV), 1)
        mask = (jnp.abs(qi - ki) <= 128) | (ki < 32) | (qi < 32)

        PART0 = (pn, pd)
        PART1 = (pd, pn)

        def mk(stage, b, bf):
            sl = pl.ds(bf * CH, CH)
            return pltpu.make_async_remote_copy(
                src_ref=comm_ref.at[2 * stage, b, sl],
                dst_ref=comm_ref.at[2 * stage + 1, b, sl],
                send_sem=send_sems.at[stage, b, bf],
                recv_sem=recv_sems.at[stage, b, bf],
                device_id=((PART0, PART1)[stage][bf],),
                device_id_type=pl.DeviceIdType.MESH,
            )

        rdma0 = [[mk(0, b, bf) for bf in range(NBF)] for b in range(B)]
        rdma1 = [[mk(1, b, bf) for bf in range(NBF)] for b in range(B)]

        barrier_waited = False
        for b in range(B):
            kb = k_ref[b].astype(jnp.bfloat16)
            vb = v_ref[b].astype(jnp.bfloat16)
            for bf in range(NBF):
                sl = pl.ds(bf * CH, CH)
                if _NO_COMPUTE:
                    part = x_ref[b, sl].astype(jnp.bfloat16)
                else:
                    mrows = mask[bf * CH:(bf + 1) * CH]
                    ctx_heads = []
                    for h in range(H_PER):
                        s = lax.dot_general(
                            q[b, bf * CH:(bf + 1) * CH, h, :], kb[:, h, :],
                            (((1,), (1,)), ((), ())),
                            preferred_element_type=jnp.float32,
                        ) * 0.125
                        w = jnp.where(mrows, jnp.exp(s), 0.0)
                        recip = 1.0 / w.sum(axis=-1, keepdims=True)
                        ctx_heads.append(
                            jnp.dot(w.astype(jnp.bfloat16), vb[:, h, :],
                                    preferred_element_type=jnp.float32)
                            * recip)
                    ctx = jnp.concatenate(ctx_heads, axis=1).astype(jnp.bfloat16)
                    part = jnp.dot(ctx, wob,
                                   preferred_element_type=jnp.float32
                                   ).astype(jnp.bfloat16)
                comm_ref[0, b, sl] = part
                if not barrier_waited:
                    pl.semaphore_wait(barrier_sem, 2)
                    barrier_waited = True
                if not _NO_COMM:
                    rdma0[b][bf].start()

        if _NO_COMM:
            for b in range(B):
                out_ref[b] = comm_ref[0, b].astype(jnp.float32)
            return

        for b in range(B):
            for bf in range(NBF):
                sl = pl.ds(bf * CH, CH)
                rdma0[b][bf].wait_recv()
                comm_ref[2, b, sl] = comm_ref[0, b, sl] + comm_ref[1, b, sl]
                rdma1[b][bf].start()

        for b in range(B):
            for bf in range(NBF):
                sl = pl.ds(bf * CH, CH)
                rdma1[b][bf].wait_recv()
                out_ref[b, sl] = (comm_ref[2, b, sl].astype(jnp.float32)
                                  + comm_ref[3, b, sl].astype(jnp.float32))

        for b in range(B):
            for bf in range(NBF):
                rdma0[b][bf].wait_send()
                rdma1[b][bf].wait_send()

    out_shape = jax.ShapeDtypeStruct((B, SQ, DM), jnp.float32)
    return pl.pallas_call(
        body,
        out_shape=out_shape,
        in_specs=[pl.BlockSpec(memory_space=pltpu.VMEM)] * 5,
        out_specs=pl.BlockSpec(memory_space=pltpu.VMEM),
        scratch_shapes=[
            pltpu.VMEM((4, B, SQ, DM), jnp.bfloat16),
            pltpu.SemaphoreType.DMA((2, B, NBF)),
            pltpu.SemaphoreType.DMA((2, B, NBF)),
        ],
        compiler_params=pltpu.CompilerParams(collective_id=0),
    )(x, Wq, K_sh, V_sh, Wo)
